# baseline (speedup 1.0000x reference)
import dataclasses
import functools

import jax
import jax.numpy as jnp
from jax import lax
from jax.experimental import pallas as pl
from jax.experimental.pallas import tpu as pltpu
from jax.experimental.pallas import tpu_sc as plsc

N_NODES = 10000
K_NEIGH = 32
D_FEAT = 128
SUBDIV = 3
SCALER = 10.0 * 1.0 * float(SUBDIV)
D_OUT = SUBDIV * D_FEAT
W_COLS = SUBDIV * K_NEIGH

NUM_CORES = 2
NUM_SUBCORES = 16
NUM_WORKERS = NUM_CORES * NUM_SUBCORES

VPW = 320
N_PAD = NUM_WORKERS * VPW
GROUP = 16
RBUF = 8
NPAIRS = VPW // GROUP
OUT_BATCH = 8

LANES = 16
FVECS = D_FEAT // LANES


def _aux_body(d_ref, o_ref):
    dist = jnp.sqrt(d_ref[...] + 1e-6)
    ws = []
    for i in range(SUBDIV):
        offset = float(i) / float(SUBDIV)
        w = jnp.exp(-SCALER * (dist - offset) ** 2)
        wsum = jnp.sum(w, axis=1, keepdims=True) + 1e-6
        ws.append(w / wsum)
    o_ref[...] = jnp.concatenate(ws, axis=-1)


def _aux_weights(distsq):
    return pl.pallas_call(
        _aux_body,
        grid=(10,),
        in_specs=[pl.BlockSpec((N_NODES // 10, K_NEIGH), lambda b: (b, 0))],
        out_specs=pl.BlockSpec((N_NODES // 10, W_COLS), lambda b: (b, 0)),
        out_shape=jax.ShapeDtypeStruct((N_NODES, W_COLS), jnp.float32),
    )(distsq)


def _sc_compiler_params():
    cp = pltpu.CompilerParams()
    if "needs_layout_passes" in pltpu.CompilerParams.__dataclass_fields__:
        cp = dataclasses.replace(cp, needs_layout_passes=False)
    return cp


def _sc_fused(features, idx_pad, aux_pad):
    mesh = plsc.VectorSubcoreMesh(core_axis_name="c", subcore_axis_name="s")

    @functools.partial(
        pl.kernel,
        compiler_params=_sc_compiler_params(),
        out_type=jax.ShapeDtypeStruct((N_NODES, D_OUT), jnp.float32),
        mesh=mesh,
        scratch_types=[
            pltpu.VMEM((VPW, K_NEIGH), jnp.int32),
            pltpu.VMEM((VPW, W_COLS), jnp.float32),
            pltpu.VMEM((RBUF, K_NEIGH, D_FEAT), jnp.float32),
            pltpu.VMEM((2, OUT_BATCH, D_OUT), jnp.float32),
            pltpu.SemaphoreType.DMA((RBUF,)),
            pltpu.SemaphoreType.DMA,
            pltpu.SemaphoreType.DMA,
        ],
    )
    def fused_kernel(
        feat_hbm, idx_hbm, aux_hbm, out_hbm,
        idx_all, aux_all, rows_v, out_b, gsem, os0, os1,
    ):
        wid = lax.axis_index("s") * NUM_CORES + lax.axis_index("c")
        base = wid * VPW
        osem = (os0, os1)

        pltpu.sync_copy(idx_hbm.at[pl.ds(base, VPW)], idx_all)
        pltpu.sync_copy(aux_hbm.at[pl.ds(base, VPW)], aux_all)

        def gather(j, lv):
            return pltpu.make_async_copy(
                feat_hbm.at[idx_all.at[lv]], rows_v.at[j], gsem.at[j]
            )

        def out_copy(b, gv0):
            return pltpu.make_async_copy(
                out_b.at[b], out_hbm.at[pl.ds(gv0, OUT_BATCH)], osem[b]
            )

        def compute(j, lv):
            lv_vec = jnp.full((LANES,), lv, jnp.int32)
            zero = jnp.zeros((LANES,), jnp.float32)
            init = tuple(zero for _ in range(SUBDIV * FVECS))

            def body(k, accs):
                accs = list(accs)
                wv = [
                    plsc.load_gather(
                        aux_all,
                        [lv_vec, jnp.full((LANES,), i * K_NEIGH, jnp.int32) + k],
                    )
                    for i in range(SUBDIV)
                ]
                for fv in range(FVECS):
                    g = rows_v[j, k, pl.ds(fv * LANES, LANES)]
                    for i in range(SUBDIV):
                        accs[i * FVECS + fv] = accs[i * FVECS + fv] + wv[i] * g
                return tuple(accs)

            return lax.fori_loop(0, K_NEIGH, body, init, unroll=4)

        for j in range(RBUF):
            gather(j, j).start()

        @pl.loop(0, NPAIRS)
        def _(n2):
            for b in range(2):
                gv0 = base + n2 * GROUP + b * OUT_BATCH

                @pl.when(
                    (n2 > 0) & (gv0 - GROUP < N_NODES)
                )
                def _(b=b, gv0=gv0):
                    out_copy(b, base).wait()

                for p in range(OUT_BATCH):
                    s = b * OUT_BATCH + p
                    j = s % RBUF
                    lv = n2 * GROUP + s
                    gather(j, lv).wait()
                    accs = compute(j, lv)
                    for i in range(SUBDIV):
                        for fv in range(FVECS):
                            out_b[
                                b, p, pl.ds(i * D_FEAT + fv * LANES, LANES)
                            ] = accs[i * FVECS + fv]

                    @pl.when(lv + RBUF < VPW)
                    def _(j=j, lv=lv):
                        gather(j, lv + RBUF).start()

                @pl.when(gv0 < N_NODES)
                def _(b=b, gv0=gv0):
                    out_copy(b, gv0).start()

        for b in range(2):
            last0 = base + (NPAIRS - 1) * GROUP + b * OUT_BATCH

            @pl.when(last0 < N_NODES)
            def _(b=b):
                out_copy(b, base).wait()

    return fused_kernel(features, idx_pad, aux_pad)


def kernel(features, distsq, neighbour_indices):
    aux = _aux_weights(distsq)
    pad = N_PAD - N_NODES
    idx_pad = jnp.pad(neighbour_indices.astype(jnp.int32), ((0, pad), (0, 0)))
    aux_pad = jnp.pad(aux, ((0, pad), (0, 0)))
    return _sc_fused(features, idx_pad, aux_pad)


# speedup vs baseline: 1.0016x; 1.0016x over previous
import dataclasses
import functools

import jax
import jax.numpy as jnp
from jax import lax
from jax.experimental import pallas as pl
from jax.experimental.pallas import tpu as pltpu
from jax.experimental.pallas import tpu_sc as plsc

N_NODES = 10000
K_NEIGH = 32
D_FEAT = 128
SUBDIV = 3
SCALER = 10.0 * 1.0 * float(SUBDIV)
D_OUT = SUBDIV * D_FEAT
W_COLS = SUBDIV * K_NEIGH

NUM_CORES = 2
NUM_SUBCORES = 16
NUM_WORKERS = NUM_CORES * NUM_SUBCORES

VPW = 320
N_PAD = NUM_WORKERS * VPW
GROUP = 16
RBUF = 8
NPAIRS = VPW // GROUP
OUT_BATCH = 8

LANES = 16
FVECS = D_FEAT // LANES


def _aux_body(d_ref, o_ref):
    dist = jnp.sqrt(d_ref[...] + 1e-6)
    ws = []
    for i in range(SUBDIV):
        offset = float(i) / float(SUBDIV)
        w = jnp.exp(-SCALER * (dist - offset) ** 2)
        wsum = jnp.sum(w, axis=1, keepdims=True) + 1e-6
        ws.append(w / wsum)
    o_ref[...] = jnp.concatenate(ws, axis=-1)


def _aux_weights(distsq):
    return pl.pallas_call(
        _aux_body,
        grid=(10,),
        in_specs=[pl.BlockSpec((N_NODES // 10, K_NEIGH), lambda b: (b, 0))],
        out_specs=pl.BlockSpec((N_NODES // 10, W_COLS), lambda b: (b, 0)),
        out_shape=jax.ShapeDtypeStruct((N_NODES, W_COLS), jnp.float32),
    )(distsq)


def _sc_compiler_params():
    cp = pltpu.CompilerParams()
    if "needs_layout_passes" in pltpu.CompilerParams.__dataclass_fields__:
        cp = dataclasses.replace(cp, needs_layout_passes=False)
    return cp


def _sc_fused(features, idx_pad, aux_pad):
    mesh = plsc.VectorSubcoreMesh(core_axis_name="c", subcore_axis_name="s")

    @functools.partial(
        pl.kernel,
        compiler_params=_sc_compiler_params(),
        out_type=jax.ShapeDtypeStruct((N_NODES, D_OUT), jnp.float32),
        mesh=mesh,
        scratch_types=[
            pltpu.VMEM((VPW, K_NEIGH), jnp.int32),
            pltpu.VMEM((VPW, W_COLS), jnp.float32),
            pltpu.VMEM((RBUF, K_NEIGH, D_FEAT), jnp.float32),
            pltpu.VMEM((2, OUT_BATCH, D_OUT), jnp.float32),
            pltpu.SemaphoreType.DMA((RBUF,)),
            pltpu.SemaphoreType.DMA,
            pltpu.SemaphoreType.DMA,
        ],
    )
    def fused_kernel(
        feat_hbm, idx_hbm, aux_hbm, out_hbm,
        idx_all, aux_all, rows_v, out_b, gsem, os0, os1,
    ):
        wid = lax.axis_index("s") * NUM_CORES + lax.axis_index("c")
        base = wid * VPW
        osem = (os0, os1)

        pltpu.sync_copy(idx_hbm.at[pl.ds(base, VPW)], idx_all)
        pltpu.sync_copy(aux_hbm.at[pl.ds(base, VPW)], aux_all)

        def gather(j, lv):
            return pltpu.make_async_copy(
                feat_hbm.at[idx_all.at[lv]], rows_v.at[j], gsem.at[j]
            )

        def out_copy(b, gv0):
            return pltpu.make_async_copy(
                out_b.at[b], out_hbm.at[pl.ds(gv0, OUT_BATCH)], osem[b]
            )

        def compute(j, lv):
            lv_vec = jnp.full((LANES,), lv, jnp.int32)
            zero = jnp.zeros((LANES,), jnp.float32)
            half_n = FVECS // 2
            accs_out = [[None] * FVECS for _ in range(SUBDIV)]
            for half in range(2):
                init = tuple(zero for _ in range(SUBDIV * half_n))

                def body(k, accs, half=half):
                    accs = list(accs)
                    wv = [
                        plsc.load_gather(
                            aux_all,
                            [
                                lv_vec,
                                jnp.full((LANES,), i * K_NEIGH, jnp.int32) + k,
                            ],
                        )
                        for i in range(SUBDIV)
                    ]
                    for fi in range(half_n):
                        fv = half * half_n + fi
                        g = rows_v[j, k, pl.ds(fv * LANES, LANES)]
                        for i in range(SUBDIV):
                            accs[i * half_n + fi] = (
                                accs[i * half_n + fi] + wv[i] * g
                            )
                    return tuple(accs)

                accs = lax.fori_loop(0, K_NEIGH, body, init, unroll=4)
                for i in range(SUBDIV):
                    for fi in range(half_n):
                        accs_out[i][half * half_n + fi] = accs[i * half_n + fi]
            return accs_out

        for j in range(RBUF):
            gather(j, j).start()

        @pl.loop(0, NPAIRS)
        def _(n2):
            for b in range(2):
                gv0 = base + n2 * GROUP + b * OUT_BATCH

                @pl.when(
                    (n2 > 0) & (gv0 - GROUP < N_NODES)
                )
                def _(b=b, gv0=gv0):
                    out_copy(b, base).wait()

                for p in range(OUT_BATCH):
                    s = b * OUT_BATCH + p
                    j = s % RBUF
                    lv = n2 * GROUP + s
                    gather(j, lv).wait()
                    accs = compute(j, lv)
                    for i in range(SUBDIV):
                        for fv in range(FVECS):
                            out_b[
                                b, p, pl.ds(i * D_FEAT + fv * LANES, LANES)
                            ] = accs[i][fv]

                    @pl.when(lv + RBUF < VPW)
                    def _(j=j, lv=lv):
                        gather(j, lv + RBUF).start()

                @pl.when(gv0 < N_NODES)
                def _(b=b, gv0=gv0):
                    out_copy(b, gv0).start()

        for b in range(2):
            last0 = base + (NPAIRS - 1) * GROUP + b * OUT_BATCH

            @pl.when(last0 < N_NODES)
            def _(b=b):
                out_copy(b, base).wait()

    return fused_kernel(features, idx_pad, aux_pad)


def kernel(features, distsq, neighbour_indices):
    aux = _aux_weights(distsq)
    pad = N_PAD - N_NODES
    idx_pad = jnp.pad(neighbour_indices.astype(jnp.int32), ((0, pad), (0, 0)))
    aux_pad = jnp.pad(aux, ((0, pad), (0, 0)))
    return _sc_fused(features, idx_pad, aux_pad)
